# baseline (speedup 1.0000x reference)
import functools

import jax
import jax.numpy as jnp
from jax import lax
from jax.experimental import pallas as pl
from jax.experimental.pallas import tpu as pltpu
from jax.experimental.pallas import tpu_sc as plsc

L = 16
NC = 2
NS = 16
NW = NC * NS
B, C, H, W = 16, 3, 512, 512
PLANE = H * W
M_TOT = B * PLANE
CHUNK = M_TOT // NW
T = 8192
N_SUB = CHUNK // T
THRESH = 0.01


@functools.partial(
    pl.kernel,
    out_type=jax.ShapeDtypeStruct((2, NW, L), jnp.float32),
    mesh=plsc.VectorSubcoreMesh(core_axis_name="c", subcore_axis_name="s"),
    scratch_types=[
        pltpu.VMEM((T,), jnp.float32),
        pltpu.VMEM((T,), jnp.float32),
        pltpu.VMEM((T,), jnp.float32),
        pltpu.VMEM((T,), jnp.float32),
        pltpu.VMEM((T,), jnp.float32),
        pltpu.VMEM((T,), jnp.float32),
        pltpu.VMEM((T,), jnp.float32),
        pltpu.VMEM((L,), jnp.float32),
        pltpu.VMEM((L,), jnp.float32),
    ],
)
def _masked_l1_partials(in_hbm, tg_hbm, mk_hbm, out_hbm,
                        mk_b, in_b0, in_b1, in_b2, tg_b0, tg_b1, tg_b2,
                        sum_b, cnt_b):
    in_b = (in_b0, in_b1, in_b2)
    tg_b = (tg_b0, tg_b1, tg_b2)
    cid = lax.axis_index("c")
    sid = lax.axis_index("s")
    wid = sid * NC + cid
    mb = wid * CHUNK
    b = mb // PLANE
    s0 = mb % PLANE

    def subtile(j, carry):
        acc, cnt = carry
        soff = s0 + j * T
        pltpu.sync_copy(mk_hbm.at[pl.ds(mb + j * T, T)], mk_b)
        for c in range(C):
            base = (b * C + c) * PLANE + soff
            pltpu.sync_copy(in_hbm.at[pl.ds(base, T)], in_b[c])
            pltpu.sync_copy(tg_hbm.at[pl.ds(base, T)], tg_b[c])

        def step(i, carry2):
            acc2, cnt2 = carry2
            sl = pl.ds(i * L, L)
            m = mk_b[sl] > THRESH
            cnt2 = cnt2 + jnp.where(m, 1.0, 0.0)
            for c in range(C):
                d = jnp.abs(in_b[c][sl] - tg_b[c][sl])
                acc2 = acc2 + jnp.where(m, d, 0.0)
            return acc2, cnt2

        return lax.fori_loop(0, T // L, step, (acc, cnt))

    z = jnp.zeros((L,), jnp.float32)
    acc, cnt = lax.fori_loop(0, N_SUB, subtile, (z, z))
    sum_b[...] = acc
    cnt_b[...] = cnt
    pltpu.sync_copy(sum_b, out_hbm.at[0, wid])
    pltpu.sync_copy(cnt_b, out_hbm.at[1, wid])


def kernel(input, target, mask):
    parts = _masked_l1_partials(
        input.reshape(-1), target.reshape(-1), mask.reshape(-1))
    sel_sum = jnp.sum(parts[0])
    count = C * jnp.sum(parts[1])
    return sel_sum / jnp.maximum(count, 1.0)


# speedup vs baseline: 1.5294x; 1.5294x over previous
import functools

import jax
import jax.numpy as jnp
from jax import lax
from jax.experimental import pallas as pl
from jax.experimental.pallas import tpu as pltpu
from jax.experimental.pallas import tpu_sc as plsc

L = 16
NC = 2
NS = 16
NW = NC * NS
B, C, H, W = 16, 3, 512, 512
PLANE = H * W
M_TOT = B * PLANE
CHUNK = M_TOT // NW
T = 8192
N_SUB = CHUNK // T
U = 4
THRESH = 0.01

_SCRATCH = (
    [pltpu.VMEM((T,), jnp.float32) for _ in range(14)]
    + [pltpu.VMEM((L,), jnp.float32) for _ in range(2)]
    + [pltpu.SemaphoreType.DMA for _ in range(2)]
)


@functools.partial(
    pl.kernel,
    out_type=jax.ShapeDtypeStruct((2, NW, L), jnp.float32),
    mesh=plsc.VectorSubcoreMesh(core_axis_name="c", subcore_axis_name="s"),
    scratch_types=_SCRATCH,
)
def _masked_l1_partials(in_hbm, tg_hbm, mk_hbm, out_hbm, *s):
    bufs = (s[0:7], s[7:14])
    sum_b, cnt_b = s[14], s[15]
    sems = (s[16], s[17])

    cid = lax.axis_index("c")
    sid = lax.axis_index("s")
    wid = sid * NC + cid
    mb = wid * CHUNK
    b = mb // PLANE
    s0 = mb % PLANE

    def issue(j, slot):
        sem = sems[slot]
        mk_b, i0, i1, i2, t0, t1, t2 = bufs[slot]
        descs = [pltpu.async_copy(mk_hbm.at[pl.ds(mb + j * T, T)], mk_b, sem)]
        for c, (ib, tb) in enumerate(((i0, t0), (i1, t1), (i2, t2))):
            base = (b * C + c) * PLANE + s0 + j * T
            descs.append(pltpu.async_copy(in_hbm.at[pl.ds(base, T)], ib, sem))
            descs.append(pltpu.async_copy(tg_hbm.at[pl.ds(base, T)], tb, sem))
        return descs

    acc = jnp.zeros((L,), jnp.float32)
    cnt = jnp.zeros((L,), jnp.float32)
    pending = {0: issue(0, 0)}
    for j in range(N_SUB):
        slot = j % 2
        if j + 1 < N_SUB:
            pending[j + 1] = issue(j + 1, 1 - slot)
        for dsc in pending.pop(j):
            dsc.wait()
        mk_b, i0, i1, i2, t0, t1, t2 = bufs[slot]

        def step(i, carry, _bufs=(mk_b, i0, i1, i2, t0, t1, t2)):
            acc2, cnt2 = carry
            mk_v, a0, a1, a2, b0, b1, b2 = _bufs
            for u in range(U):
                sl = pl.ds((i * U + u) * L, L)
                m = mk_v[sl] > THRESH
                cnt2 = cnt2 + jnp.where(m, 1.0, 0.0)
                for av, bv in ((a0, b0), (a1, b1), (a2, b2)):
                    d = jnp.abs(av[sl] - bv[sl])
                    acc2 = acc2 + jnp.where(m, d, 0.0)
            return acc2, cnt2

        acc, cnt = lax.fori_loop(0, T // (L * U), step, (acc, cnt))

    sum_b[...] = acc
    cnt_b[...] = cnt
    pltpu.sync_copy(sum_b, out_hbm.at[0, wid])
    pltpu.sync_copy(cnt_b, out_hbm.at[1, wid])


def kernel(input, target, mask):
    parts = _masked_l1_partials(
        input.reshape(-1), target.reshape(-1), mask.reshape(-1))
    sel_sum = jnp.sum(parts[0])
    count = C * jnp.sum(parts[1])
    return sel_sum / jnp.maximum(count, 1.0)


# speedup vs baseline: 1.5431x; 1.0090x over previous
import functools

import jax
import jax.numpy as jnp
from jax import lax
from jax.experimental import pallas as pl
from jax.experimental.pallas import tpu as pltpu
from jax.experimental.pallas import tpu_sc as plsc

L = 16
NC = 2
NS = 16
NW = NC * NS
B, C, H, W = 16, 3, 512, 512
PLANE = H * W
M_TOT = B * PLANE
CHUNK = M_TOT // NW
T = 8192
N_SUB = CHUNK // T
U = 4
THRESH = 0.01

_SCRATCH = (
    [pltpu.VMEM((T,), jnp.float32) for _ in range(14)]
    + [pltpu.VMEM((L,), jnp.float32) for _ in range(2)]
    + [pltpu.SemaphoreType.DMA for _ in range(2)]
)


@functools.partial(
    pl.kernel,
    out_type=jax.ShapeDtypeStruct((2, NW, L), jnp.float32),
    mesh=plsc.VectorSubcoreMesh(core_axis_name="c", subcore_axis_name="s"),
    scratch_types=_SCRATCH,
)
def _masked_l1_partials(in_hbm, tg_hbm, mk_hbm, out_hbm, *s):
    bufs = (s[0:7], s[7:14])
    sum_b, cnt_b = s[14], s[15]
    sems = (s[16], s[17])

    cid = lax.axis_index("c")
    sid = lax.axis_index("s")
    wid = sid * NC + cid
    mb = wid * CHUNK
    b = mb // PLANE
    s0 = mb % PLANE

    def issue(j, slot):
        sem = sems[slot]
        mk_b, i0, i1, i2, t0, t1, t2 = bufs[slot]
        descs = [pltpu.async_copy(mk_hbm.at[pl.ds(mb + j * T, T)], mk_b, sem)]
        for c, (ib, tb) in enumerate(((i0, t0), (i1, t1), (i2, t2))):
            base = (b * C + c) * PLANE + s0 + j * T
            descs.append(pltpu.async_copy(in_hbm.at[pl.ds(base, T)], ib, sem))
            descs.append(pltpu.async_copy(tg_hbm.at[pl.ds(base, T)], tb, sem))
        return descs

    z = jnp.zeros((L,), jnp.float32)
    carry0 = (z,) * 8
    pending = {0: issue(0, 0)}
    for j in range(N_SUB):
        slot = j % 2
        if j + 1 < N_SUB:
            pending[j + 1] = issue(j + 1, 1 - slot)
        for dsc in pending.pop(j):
            dsc.wait()
        mk_b, i0, i1, i2, t0, t1, t2 = bufs[slot]

        def step(i, carry, _bufs=(mk_b, i0, i1, i2, t0, t1, t2)):
            accs = list(carry)
            mk_v, a0, a1, a2, b0, b1, b2 = _bufs
            for u in range(U):
                p = u % 2
                sl = pl.ds((i * U + u) * L, L)
                m = mk_v[sl] > THRESH
                accs[6 + p] = accs[6 + p] + jnp.where(m, 1.0, 0.0)
                for c, (av, bv) in enumerate(((a0, b0), (a1, b1), (a2, b2))):
                    d = jnp.abs(av[sl] - bv[sl])
                    accs[2 * c + p] = accs[2 * c + p] + jnp.where(m, d, 0.0)
            return tuple(accs)

        carry0 = lax.fori_loop(0, T // (L * U), step, carry0)

    sum_b[...] = (carry0[0] + carry0[1]) + (carry0[2] + carry0[3]) \
        + (carry0[4] + carry0[5])
    cnt_b[...] = carry0[6] + carry0[7]
    pltpu.sync_copy(sum_b, out_hbm.at[0, wid])
    pltpu.sync_copy(cnt_b, out_hbm.at[1, wid])


def kernel(input, target, mask):
    parts = _masked_l1_partials(
        input.reshape(-1), target.reshape(-1), mask.reshape(-1))
    sel_sum = jnp.sum(parts[0])
    count = C * jnp.sum(parts[1])
    return sel_sum / jnp.maximum(count, 1.0)


# speedup vs baseline: 3.4820x; 2.2565x over previous
import functools

import jax
import jax.numpy as jnp
from jax import lax
from jax.experimental import pallas as pl
from jax.experimental.pallas import tpu as pltpu
from jax.experimental.pallas import tpu_sc as plsc

L = 16
NC = 2
NS = 16
NW = NC * NS
B, C, H, W = 16, 3, 512, 512
PLANE = H * W
M_TOT = B * PLANE
CHUNK = M_TOT // NW
T = 8192
N_SUB = CHUNK // T
U = 4
W2D = 512
THRESH = 0.01

_SCRATCH = (
    [pltpu.VMEM((T // W2D, W2D), jnp.float32) for _ in range(14)]
    + [pltpu.VMEM((L,), jnp.float32) for _ in range(2)]
    + [pltpu.SemaphoreType.DMA for _ in range(2)]
)


@functools.partial(
    pl.kernel,
    out_type=jax.ShapeDtypeStruct((2, NW, L), jnp.float32),
    mesh=plsc.VectorSubcoreMesh(core_axis_name="c", subcore_axis_name="s"),
    scratch_types=_SCRATCH,
)
def _masked_l1_partials(in_hbm, tg_hbm, mk_hbm, out_hbm, *s):
    bufs = (s[0:7], s[7:14])
    sum_b, cnt_b = s[14], s[15]
    sems = (s[16], s[17])

    cid = lax.axis_index("c")
    sid = lax.axis_index("s")
    wid = sid * NC + cid
    mb = wid * CHUNK
    b = mb // PLANE
    s0 = mb % PLANE

    RT = T // W2D

    def issue(j, slot):
        sem = sems[slot]
        mk_b, i0, i1, i2, t0, t1, t2 = bufs[slot]
        mrow = pl.multiple_of((mb + j * T) // W2D, 8)
        descs = [pltpu.async_copy(mk_hbm.at[pl.ds(mrow, RT), :], mk_b, sem)]
        for c, (ib, tb) in enumerate(((i0, t0), (i1, t1), (i2, t2))):
            row = pl.multiple_of(((b * C + c) * PLANE + s0 + j * T) // W2D, 8)
            descs.append(pltpu.async_copy(in_hbm.at[pl.ds(row, RT), :], ib, sem))
            descs.append(pltpu.async_copy(tg_hbm.at[pl.ds(row, RT), :], tb, sem))
        return descs

    z = jnp.zeros((L,), jnp.float32)
    carry0 = (z,) * 8
    pending = {0: issue(0, 0)}
    for j in range(N_SUB):
        slot = j % 2
        if j + 1 < N_SUB:
            pending[j + 1] = issue(j + 1, 1 - slot)
        for dsc in pending.pop(j):
            dsc.wait()
        def row_loop(r, carry, _bufs=bufs[slot]):
            def step(g, carry2):
                accs = list(carry2)
                mk_v, a0, a1, a2, b0, b1, b2 = _bufs
                for u in range(U):
                    p = u % 2
                    sl = pl.ds((g * U + u) * L, L)
                    m = mk_v[r, sl] > THRESH
                    accs[6 + p] = accs[6 + p] + jnp.where(m, 1.0, 0.0)
                    for c, (av, bv) in enumerate(((a0, b0), (a1, b1), (a2, b2))):
                        d = jnp.abs(av[r, sl] - bv[r, sl])
                        accs[2 * c + p] = accs[2 * c + p] + jnp.where(m, d, 0.0)
                return tuple(accs)

            return lax.fori_loop(0, W2D // (L * U), step, carry)

        carry0 = lax.fori_loop(0, RT, row_loop, carry0)

    sum_b[...] = (carry0[0] + carry0[1]) + (carry0[2] + carry0[3]) \
        + (carry0[4] + carry0[5])
    cnt_b[...] = carry0[6] + carry0[7]
    pltpu.sync_copy(sum_b, out_hbm.at[0, wid])
    pltpu.sync_copy(cnt_b, out_hbm.at[1, wid])


def kernel(input, target, mask):
    parts = _masked_l1_partials(
        input.reshape(B * C * H, W),
        target.reshape(B * C * H, W),
        mask.reshape(B * H, W))
    sel_sum = jnp.sum(parts[0])
    count = C * jnp.sum(parts[1])
    return sel_sum / jnp.maximum(count, 1.0)


# speedup vs baseline: 4.2107x; 1.2093x over previous
import functools

import jax
import jax.numpy as jnp
from jax import lax
from jax.experimental import pallas as pl
from jax.experimental.pallas import tpu as pltpu
from jax.experimental.pallas import tpu_sc as plsc

L = 16
NC = 2
NS = 16
NW = NC * NS
B, C, H, W = 16, 3, 512, 512
PLANE = H * W
W2D = 512
T = 8192
U = 4
THRESH = 0.01

K_SC = 6
M_SC = K_SC * PLANE
CHUNK = M_SC // NW
N_SUB = CHUNK // T
RT = T // W2D

_SCRATCH = (
    [pltpu.VMEM((RT, W2D), jnp.float32) for _ in range(14)]
    + [pltpu.VMEM((L,), jnp.float32) for _ in range(2)]
    + [pltpu.SemaphoreType.DMA for _ in range(2)]
)


@functools.partial(
    pl.kernel,
    out_type=jax.ShapeDtypeStruct((2, NW, L), jnp.float32),
    mesh=plsc.VectorSubcoreMesh(core_axis_name="c", subcore_axis_name="s"),
    scratch_types=_SCRATCH,
)
def _sc_partials(in_hbm, tg_hbm, mk_hbm, out_hbm, *s):
    bufs = (s[0:7], s[7:14])
    sum_b, cnt_b = s[14], s[15]
    sems = (s[16], s[17])

    cid = lax.axis_index("c")
    sid = lax.axis_index("s")
    wid = sid * NC + cid
    mb = wid * CHUNK

    def issue(j, slot):
        sem = sems[slot]
        mk_b, i0, i1, i2, t0, t1, t2 = bufs[slot]
        moff = mb + j * T
        bb = moff // PLANE
        q = moff % PLANE
        mrow = pl.multiple_of(moff // W2D, 8)
        descs = [pltpu.async_copy(mk_hbm.at[pl.ds(mrow, RT), :], mk_b, sem)]
        for c, (ib, tb) in enumerate(((i0, t0), (i1, t1), (i2, t2))):
            row = pl.multiple_of(((bb * C + c) * PLANE + q) // W2D, 8)
            descs.append(pltpu.async_copy(in_hbm.at[pl.ds(row, RT), :], ib, sem))
            descs.append(pltpu.async_copy(tg_hbm.at[pl.ds(row, RT), :], tb, sem))
        return descs

    z = jnp.zeros((L,), jnp.float32)
    carry0 = (z,) * 8
    pending = {0: issue(0, 0)}
    for j in range(N_SUB):
        slot = j % 2
        if j + 1 < N_SUB:
            pending[j + 1] = issue(j + 1, 1 - slot)
        for dsc in pending.pop(j):
            dsc.wait()

        def row_loop(r, carry, _bufs=bufs[slot]):
            def step(g, carry2):
                accs = list(carry2)
                mk_v, a0, a1, a2, b0, b1, b2 = _bufs
                for u in range(U):
                    p = u % 2
                    sl = pl.ds((g * U + u) * L, L)
                    m = mk_v[r, sl] > THRESH
                    accs[6 + p] = accs[6 + p] + jnp.where(m, 1.0, 0.0)
                    for c, (av, bv) in enumerate(((a0, b0), (a1, b1), (a2, b2))):
                        d = jnp.abs(av[r, sl] - bv[r, sl])
                        accs[2 * c + p] = accs[2 * c + p] + jnp.where(m, d, 0.0)
                return tuple(accs)

            return lax.fori_loop(0, W2D // (L * U), step, carry)

        carry0 = lax.fori_loop(0, RT, row_loop, carry0)

    sum_b[...] = (carry0[0] + carry0[1]) + (carry0[2] + carry0[3]) \
        + (carry0[4] + carry0[5])
    cnt_b[...] = carry0[6] + carry0[7]
    pltpu.sync_copy(sum_b, out_hbm.at[0, wid])
    pltpu.sync_copy(cnt_b, out_hbm.at[1, wid])


def _tc_body(in_ref, tg_ref, mk_ref, out_ref):
    g = pl.program_id(0)
    m = mk_ref[0, 0] > THRESH
    diff = jnp.abs(in_ref[0] - tg_ref[0])
    masked = jnp.where(m[None], diff, 0.0)
    psum = jnp.sum(masked.reshape(C * H // 8, 8, W), axis=0)
    pcnt = jnp.sum(jnp.where(m, 1.0, 0.0).reshape(H // 8, 8, W), axis=0)

    @pl.when(g == 0)
    def _init():
        out_ref[0] = psum
        out_ref[1] = pcnt

    @pl.when(g > 0)
    def _accum():
        out_ref[0] += psum
        out_ref[1] += pcnt


def _tc_partials(input, target, mask):
    return pl.pallas_call(
        _tc_body,
        grid=(B - K_SC,),
        in_specs=[
            pl.BlockSpec((1, C, H, W), lambda g: (g + K_SC, 0, 0, 0)),
            pl.BlockSpec((1, C, H, W), lambda g: (g + K_SC, 0, 0, 0)),
            pl.BlockSpec((1, 1, H, W), lambda g: (g + K_SC, 0, 0, 0)),
        ],
        out_specs=pl.BlockSpec((2, 8, W), lambda g: (0, 0, 0)),
        out_shape=jax.ShapeDtypeStruct((2, 8, W), jnp.float32),
    )(input, target, mask)


def kernel(input, target, mask):
    sc = _sc_partials(
        input.reshape(B * C * H, W),
        target.reshape(B * C * H, W),
        mask.reshape(B * H, W))
    tc = _tc_partials(input, target, mask)
    sel_sum = jnp.sum(sc[0]) + jnp.sum(tc[0])
    count = C * (jnp.sum(sc[1]) + jnp.sum(tc[1]))
    return sel_sum / jnp.maximum(count, 1.0)
